# baseline (speedup 1.0000x reference)
import functools

import jax
import jax.numpy as jnp
from jax import lax
from jax.experimental import pallas as pl
from jax.experimental.pallas import tpu as pltpu
from jax.experimental.pallas import tpu_sc as plsc

N = 65536
D = 128
C = 512
MARGIN = 0.5

NC = 2
NS = 16
NW = NC * NS

S = 16384
RPW = S // NW
CH = 128
NCH = RPW // CH

BLK = 2048
TG = (N - S) // BLK

_GDN = lax.GatherDimensionNumbers(
    offset_dims=(), collapsed_slice_dims=(0,), start_index_map=(0,))


def _bcast_lane(vec, idx16):
    return lax.gather(vec, idx16[:, None], _GDN, slice_sizes=(1,),
                      mode=lax.GatherScatterMode.PROMISE_IN_BOUNDS)


def _rsqrt16(x):
    xh = x * 0.5
    i = plsc.bitcast(x, jnp.int32)
    i = jnp.int32(0x5F3759DF) - lax.shift_right_logical(i, 1)
    y = plsc.bitcast(i, jnp.float32)
    y = y * (1.5 - xh * y * y)
    y = y * (1.5 - xh * y * y)
    return y


def _sc_body(f1, f2, f3, lbl, out, cnt_out, lblv, buf0, buf1, acc, cntv,
             sem0, sem1):
    wid = lax.axis_index("s") * NC + lax.axis_index("c")
    base = wid * RPW

    iota = lax.iota(jnp.int32, 16)
    cols = [iota + (16 * k) for k in range(8)]
    bfly = [lax.bitwise_xor(iota, jnp.int32(m)) for m in (1, 2, 4, 8)]
    zeros16 = jnp.zeros((16,), jnp.float32)
    ones16 = jnp.ones((16,), jnp.float32)
    lane0 = iota == 0

    pltpu.sync_copy(lbl.at[pl.ds(base, RPW)], lblv)

    @plsc.parallel_loop(0, C // 16, unroll=4)
    def _zero_cnt(r):
        cntv[pl.ds(r * 16, 16)] = zeros16

    for f, feat in enumerate((f1, f2, f3)):
        @plsc.parallel_loop(0, C, unroll=4)
        def _zero(r):
            for k in range(8):
                acc[r, pl.ds(16 * k, 16)] = zeros16

        pltpu.async_copy(feat.at[pl.ds(base, CH), :], buf0, sem0)
        pltpu.async_copy(feat.at[pl.ds(base + CH, CH), :], buf1, sem1)

        @pl.loop(0, NCH, step=2)
        def _chunks(ch):
            for b, (buf, sem) in enumerate(((buf0, sem0), (buf1, sem1))):
                cur = ch + b
                pltpu.make_async_copy(
                    feat.at[pl.ds(base, CH), :], buf, sem).wait()

                @plsc.parallel_loop(0, CH, unroll=16)
                def _rows(ri):
                    lbl16 = lblv[pl.ds(cur * CH + (ri // 16) * 16, 16)]
                    lane = jnp.full((16,), ri % 16, jnp.int32)
                    v = [buf[ri, pl.ds(16 * k, 16)] for k in range(8)]
                    ss = v[0] * v[0]
                    for k in range(1, 8):
                        ss = v[k] * v[k] + ss
                    for perm in bfly:
                        ss = ss + _bcast_lane(ss, perm)
                    inv = _rsqrt16(ss)
                    row = _bcast_lane(lbl16, lane)
                    for k in range(8):
                        plsc.addupdate_scatter(acc, [row, cols[k]],
                                               v[k] * inv)
                    if f == 0:
                        plsc.addupdate_scatter(cntv, [row], ones16,
                                               mask=lane0)

                nxt = cur + 2

                @pl.when(nxt < NCH)
                def _prefetch():
                    pltpu.async_copy(
                        feat.at[pl.ds(base + nxt * CH, CH), :], buf, sem)

        pltpu.sync_copy(acc, out.at[f, wid])
        if f == 0:
            pltpu.sync_copy(cntv, cnt_out.at[wid, 0])


_sc_call = functools.partial(
    pl.kernel,
    mesh=plsc.VectorSubcoreMesh(core_axis_name="c", subcore_axis_name="s"),
    compiler_params=pltpu.CompilerParams(needs_layout_passes=False),
    out_type=(
        jax.ShapeDtypeStruct((3, NW, C, D), jnp.float32),
        jax.ShapeDtypeStruct((NW, 1, C), jnp.float32),
    ),
    scratch_types=[
        pltpu.VMEM((RPW,), jnp.int32),
        pltpu.VMEM((CH, D), jnp.float32),
        pltpu.VMEM((CH, D), jnp.float32),
        pltpu.VMEM((C, D), jnp.float32),
        pltpu.VMEM((C,), jnp.float32),
        pltpu.SemaphoreType.DMA,
        pltpu.SemaphoreType.DMA,
    ],
)(_sc_body)


def _tc_main_body(lbl_ref, f1_ref, f2_ref, f3_ref, sums_ref, cnt_ref,
                  acc_s, cnt_s):
    i = pl.program_id(0)

    lbl = lbl_ref[0, 0, :]

    raw = jnp.concatenate(
        [f1_ref[...], f2_ref[...], f3_ref[...]], axis=1)

    jrow = jax.lax.broadcasted_iota(jnp.int32, (3 * D, 8), 0) // D
    jcol = jax.lax.broadcasted_iota(jnp.int32, (3 * D, 8), 1)
    bdiag = (jrow == jcol).astype(jnp.float32)
    pad = (jax.lax.broadcasted_iota(jnp.int32, (BLK, 8), 1) >= 3
           ).astype(jnp.float32)
    ssq = jax.lax.dot_general(
        raw * raw, bdiag, (((1,), (0,)), ((), ())),
        preferred_element_type=jnp.float32) + pad
    inv = jax.lax.rsqrt(ssq)
    inv_full = jax.lax.dot_general(
        inv, bdiag, (((1,), (1,)), ((), ())),
        preferred_element_type=jnp.float32)
    fstack = raw * inv_full

    classes = jax.lax.broadcasted_iota(jnp.int32, (BLK, C), 1)
    onehot = (lbl[:, None] == classes).astype(jnp.float32)

    part = jax.lax.dot_general(
        onehot, fstack, (((0,), (0,)), ((), ())),
        preferred_element_type=jnp.float32)
    ones8 = jnp.ones((8, BLK), jnp.float32)
    cnt = jax.lax.dot_general(
        ones8, onehot, (((1,), (0,)), ((), ())),
        preferred_element_type=jnp.float32)[0:1, :]

    @pl.when(i == 0)
    def _init():
        acc_s[...] = part
        cnt_s[...] = cnt

    @pl.when(i > 0)
    def _acc():
        acc_s[...] += part
        cnt_s[...] += cnt

    @pl.when(i == TG - 1)
    def _emit():
        sums_ref[...] = acc_s[...]
        cnt_ref[...] = cnt_s[...]


FB = 8
FG = NW // FB


def _tc_finish_body(csc_ref, p_ref, tsum_ref, tcnt_ref, out_ref,
                    acc_ref, cnt_ref):
    i = pl.program_id(0)
    part = jnp.sum(p_ref[...], axis=1)
    cnt = jnp.sum(csc_ref[:, 0, :], axis=0)[None, :]

    @pl.when(i == 0)
    def _init():
        t = tsum_ref[...]
        acc_ref[0] = part[0] + t[:, 0:D]
        acc_ref[1] = part[1] + t[:, D:2 * D]
        acc_ref[2] = part[2] + t[:, 2 * D:3 * D]
        cnt_ref[...] = cnt + tcnt_ref[...]

    @pl.when(i > 0)
    def _acc():
        acc_ref[...] += part
        cnt_ref[...] += cnt

    @pl.when(i == FG - 1)
    def _finish():
        counts = cnt_ref[0, :]
        denom = jnp.maximum(counts, 1.0)[:, None]
        c1 = acc_ref[0] / denom
        c2 = acc_ref[1] / denom
        c3 = acc_ref[2] / denom
        d = (jnp.sum((c1 - c2) ** 2, axis=1)
             + jnp.sum((c1 - c3) ** 2, axis=1)
             + jnp.sum((c2 - c3) ** 2, axis=1))
        per_class = jnp.where(counts > 0.0, jnp.maximum(d - MARGIN, 0.0), 0.0)
        out_ref[...] = jnp.sum(per_class)[None, None]


@jax.jit
def kernel(feat1, feat2, feat3, label1):
    lbl = label1.astype(jnp.int32)

    s_blk = S // BLK
    lbl3 = lbl.reshape(N // BLK, 1, BLK)
    tc_sums, tc_cnt = pl.pallas_call(
        _tc_main_body,
        grid=(TG,),
        in_specs=[
            pl.BlockSpec((1, 1, BLK), lambda i: (i + s_blk, 0, 0)),
            pl.BlockSpec((BLK, D), lambda i: (i + s_blk, 0)),
            pl.BlockSpec((BLK, D), lambda i: (i + s_blk, 0)),
            pl.BlockSpec((BLK, D), lambda i: (i + s_blk, 0)),
        ],
        out_specs=[
            pl.BlockSpec((C, 3 * D), lambda i: (0, 0)),
            pl.BlockSpec((1, C), lambda i: (0, 0)),
        ],
        out_shape=[
            jax.ShapeDtypeStruct((C, 3 * D), jnp.float32),
            jax.ShapeDtypeStruct((1, C), jnp.float32),
        ],
        scratch_shapes=[
            pltpu.VMEM((C, 3 * D), jnp.float32),
            pltpu.VMEM((1, C), jnp.float32),
        ],
    )(lbl3, feat1, feat2, feat3)

    partials, cnts = _sc_call(feat1, feat2, feat3, lbl)

    out = pl.pallas_call(
        _tc_finish_body,
        grid=(FG,),
        in_specs=[
            pl.BlockSpec((FB, 1, C), lambda i: (i, 0, 0)),
            pl.BlockSpec((3, FB, C, D), lambda i: (0, i, 0, 0)),
            pl.BlockSpec((C, 3 * D), lambda i: (0, 0)),
            pl.BlockSpec((1, C), lambda i: (0, 0)),
        ],
        out_specs=pl.BlockSpec((1, 1), lambda i: (0, 0)),
        out_shape=jax.ShapeDtypeStruct((1, 1), jnp.float32),
        scratch_shapes=[
            pltpu.VMEM((3, C, D), jnp.float32),
            pltpu.VMEM((1, C), jnp.float32),
        ],
    )(cnts, partials, tc_sums, tc_cnt)
    return out[0, 0]


# speedup vs baseline: 1.0738x; 1.0738x over previous
import functools

import jax
import jax.numpy as jnp
from jax import lax
from jax.experimental import pallas as pl
from jax.experimental.pallas import tpu as pltpu
from jax.experimental.pallas import tpu_sc as plsc

N = 65536
D = 128
C = 512
MARGIN = 0.5

NC = 2
NS = 16
NW = NC * NS

S = 16384
RPW = S // NW
CH = 128
NCH = RPW // CH

BLK = 4096
TG = (N - S) // BLK

_GDN = lax.GatherDimensionNumbers(
    offset_dims=(), collapsed_slice_dims=(0,), start_index_map=(0,))


def _bcast_lane(vec, idx16):
    return lax.gather(vec, idx16[:, None], _GDN, slice_sizes=(1,),
                      mode=lax.GatherScatterMode.PROMISE_IN_BOUNDS)


def _rsqrt16(x):
    xh = x * 0.5
    i = plsc.bitcast(x, jnp.int32)
    i = jnp.int32(0x5F3759DF) - lax.shift_right_logical(i, 1)
    y = plsc.bitcast(i, jnp.float32)
    y = y * (1.5 - xh * y * y)
    y = y * (1.5 - xh * y * y)
    return y


def _sc_body(f1, f2, f3, lbl, out, cnt_out, lblv, buf0, buf1, acc, cntv,
             sem0, sem1):
    wid = lax.axis_index("s") * NC + lax.axis_index("c")
    base = wid * RPW

    iota = lax.iota(jnp.int32, 16)
    cols = [iota + (16 * k) for k in range(8)]
    bfly = [lax.bitwise_xor(iota, jnp.int32(m)) for m in (1, 2, 4, 8)]
    zeros16 = jnp.zeros((16,), jnp.float32)
    ones16 = jnp.ones((16,), jnp.float32)
    lane0 = iota == 0

    pltpu.sync_copy(lbl.at[pl.ds(base, RPW)], lblv)

    @plsc.parallel_loop(0, C // 16, unroll=4)
    def _zero_cnt(r):
        cntv[pl.ds(r * 16, 16)] = zeros16

    for f, feat in enumerate((f1, f2, f3)):
        @plsc.parallel_loop(0, C, unroll=4)
        def _zero(r):
            for k in range(8):
                acc[r, pl.ds(16 * k, 16)] = zeros16

        pltpu.async_copy(feat.at[pl.ds(base, CH), :], buf0, sem0)
        pltpu.async_copy(feat.at[pl.ds(base + CH, CH), :], buf1, sem1)

        @pl.loop(0, NCH, step=2)
        def _chunks(ch):
            for b, (buf, sem) in enumerate(((buf0, sem0), (buf1, sem1))):
                cur = ch + b
                pltpu.make_async_copy(
                    feat.at[pl.ds(base, CH), :], buf, sem).wait()

                @plsc.parallel_loop(0, CH, unroll=16)
                def _rows(ri):
                    lbl16 = lblv[pl.ds(cur * CH + (ri // 16) * 16, 16)]
                    lane = jnp.full((16,), ri % 16, jnp.int32)
                    v = [buf[ri, pl.ds(16 * k, 16)] for k in range(8)]
                    ss = v[0] * v[0]
                    for k in range(1, 8):
                        ss = v[k] * v[k] + ss
                    for perm in bfly:
                        ss = ss + _bcast_lane(ss, perm)
                    inv = _rsqrt16(ss)
                    row = _bcast_lane(lbl16, lane)
                    for k in range(8):
                        plsc.addupdate_scatter(acc, [row, cols[k]],
                                               v[k] * inv)
                    if f == 0:
                        plsc.addupdate_scatter(cntv, [row], ones16,
                                               mask=lane0)

                nxt = cur + 2

                @pl.when(nxt < NCH)
                def _prefetch():
                    pltpu.async_copy(
                        feat.at[pl.ds(base + nxt * CH, CH), :], buf, sem)

        pltpu.sync_copy(acc, out.at[f, wid])
        if f == 0:
            pltpu.sync_copy(cntv, cnt_out.at[wid, 0])


_sc_call = functools.partial(
    pl.kernel,
    mesh=plsc.VectorSubcoreMesh(core_axis_name="c", subcore_axis_name="s"),
    compiler_params=pltpu.CompilerParams(needs_layout_passes=False),
    out_type=(
        jax.ShapeDtypeStruct((3, NW, C, D), jnp.float32),
        jax.ShapeDtypeStruct((NW, 1, C), jnp.float32),
    ),
    scratch_types=[
        pltpu.VMEM((RPW,), jnp.int32),
        pltpu.VMEM((CH, D), jnp.float32),
        pltpu.VMEM((CH, D), jnp.float32),
        pltpu.VMEM((C, D), jnp.float32),
        pltpu.VMEM((C,), jnp.float32),
        pltpu.SemaphoreType.DMA,
        pltpu.SemaphoreType.DMA,
    ],
)(_sc_body)


def _tc_main_body(lbl_ref, f1_ref, f2_ref, f3_ref, sums_ref, cnt_ref,
                  acc_s, cnt_s):
    i = pl.program_id(0)

    lbl = lbl_ref[0, 0, :]

    raw = jnp.concatenate(
        [f1_ref[...], f2_ref[...], f3_ref[...]], axis=1)

    jrow = jax.lax.broadcasted_iota(jnp.int32, (3 * D, 8), 0) // D
    jcol = jax.lax.broadcasted_iota(jnp.int32, (3 * D, 8), 1)
    bdiag = (jrow == jcol).astype(jnp.float32)
    pad = (jax.lax.broadcasted_iota(jnp.int32, (BLK, 8), 1) >= 3
           ).astype(jnp.float32)
    ssq = jax.lax.dot_general(
        raw * raw, bdiag, (((1,), (0,)), ((), ())),
        preferred_element_type=jnp.float32) + pad
    inv = jax.lax.rsqrt(ssq)
    inv_full = jax.lax.dot_general(
        inv, bdiag, (((1,), (1,)), ((), ())),
        preferred_element_type=jnp.float32)
    fstack = raw * inv_full

    classes = jax.lax.broadcasted_iota(jnp.int32, (BLK, C), 1)
    onehot = (lbl[:, None] == classes).astype(jnp.float32)

    part = jax.lax.dot_general(
        onehot, fstack, (((0,), (0,)), ((), ())),
        preferred_element_type=jnp.float32)
    ones8 = jnp.ones((8, BLK), jnp.float32)
    cnt = jax.lax.dot_general(
        ones8, onehot, (((1,), (0,)), ((), ())),
        preferred_element_type=jnp.float32)[0:1, :]

    @pl.when(i == 0)
    def _init():
        acc_s[...] = part
        cnt_s[...] = cnt

    @pl.when(i > 0)
    def _acc():
        acc_s[...] += part
        cnt_s[...] += cnt

    @pl.when(i == TG - 1)
    def _emit():
        sums_ref[...] = acc_s[...]
        cnt_ref[...] = cnt_s[...]


FB = 8
FG = NW // FB


def _tc_finish_body(csc_ref, p_ref, tsum_ref, tcnt_ref, out_ref,
                    acc_ref, cnt_ref):
    i = pl.program_id(0)
    part = jnp.sum(p_ref[...], axis=1)
    cnt = jnp.sum(csc_ref[:, 0, :], axis=0)[None, :]

    @pl.when(i == 0)
    def _init():
        t = tsum_ref[...]
        acc_ref[0] = part[0] + t[:, 0:D]
        acc_ref[1] = part[1] + t[:, D:2 * D]
        acc_ref[2] = part[2] + t[:, 2 * D:3 * D]
        cnt_ref[...] = cnt + tcnt_ref[...]

    @pl.when(i > 0)
    def _acc():
        acc_ref[...] += part
        cnt_ref[...] += cnt

    @pl.when(i == FG - 1)
    def _finish():
        counts = cnt_ref[0, :]
        denom = jnp.maximum(counts, 1.0)[:, None]
        c1 = acc_ref[0] / denom
        c2 = acc_ref[1] / denom
        c3 = acc_ref[2] / denom
        d = (jnp.sum((c1 - c2) ** 2, axis=1)
             + jnp.sum((c1 - c3) ** 2, axis=1)
             + jnp.sum((c2 - c3) ** 2, axis=1))
        per_class = jnp.where(counts > 0.0, jnp.maximum(d - MARGIN, 0.0), 0.0)
        out_ref[...] = jnp.sum(per_class)[None, None]


@jax.jit
def kernel(feat1, feat2, feat3, label1):
    lbl = label1.astype(jnp.int32)

    s_blk = S // BLK
    lbl3 = lbl.reshape(N // BLK, 1, BLK)
    tc_sums, tc_cnt = pl.pallas_call(
        _tc_main_body,
        grid=(TG,),
        in_specs=[
            pl.BlockSpec((1, 1, BLK), lambda i: (i + s_blk, 0, 0)),
            pl.BlockSpec((BLK, D), lambda i: (i + s_blk, 0)),
            pl.BlockSpec((BLK, D), lambda i: (i + s_blk, 0)),
            pl.BlockSpec((BLK, D), lambda i: (i + s_blk, 0)),
        ],
        out_specs=[
            pl.BlockSpec((C, 3 * D), lambda i: (0, 0)),
            pl.BlockSpec((1, C), lambda i: (0, 0)),
        ],
        out_shape=[
            jax.ShapeDtypeStruct((C, 3 * D), jnp.float32),
            jax.ShapeDtypeStruct((1, C), jnp.float32),
        ],
        scratch_shapes=[
            pltpu.VMEM((C, 3 * D), jnp.float32),
            pltpu.VMEM((1, C), jnp.float32),
        ],
    )(lbl3, feat1, feat2, feat3)

    partials, cnts = _sc_call(feat1, feat2, feat3, lbl)

    out = pl.pallas_call(
        _tc_finish_body,
        grid=(FG,),
        in_specs=[
            pl.BlockSpec((FB, 1, C), lambda i: (i, 0, 0)),
            pl.BlockSpec((3, FB, C, D), lambda i: (0, i, 0, 0)),
            pl.BlockSpec((C, 3 * D), lambda i: (0, 0)),
            pl.BlockSpec((1, C), lambda i: (0, 0)),
        ],
        out_specs=pl.BlockSpec((1, 1), lambda i: (0, 0)),
        out_shape=jax.ShapeDtypeStruct((1, 1), jnp.float32),
        scratch_shapes=[
            pltpu.VMEM((3, C, D), jnp.float32),
            pltpu.VMEM((1, C), jnp.float32),
        ],
    )(cnts, partials, tc_sums, tc_cnt)
    return out[0, 0]
